# Initial kernel scaffold; baseline (speedup 1.0000x reference)
import jax
import jax.numpy as jnp
from jax.experimental import pallas as pl


def kernel(features, edge_index, W_self1, W_neigh1, b1, W_self2, W_neigh2, b2):
    raise NotImplementedError("write your pallas kernel here")


# speedup vs baseline: 10.3756x; 10.3756x over previous
import functools

import jax
import jax.numpy as jnp
from jax import lax
from jax.experimental import pallas as pl
from jax.experimental.pallas import tpu as pltpu
from jax.experimental.pallas import tpu_sc as plsc

N = 10000
E = 320000
D = 128
H = 256
C = 32

NP = 10240
K = 125
ER = E // K
NC = 2
NS = 16
NW = NC * NS
CH = ER // NW
RT = NP // NS
ZR = 64

RB = 1000
GRID = N // RB
PF = 128


def _sc_agg(table, src2d, dst2d, F, with_deg):
    mesh = plsc.VectorSubcoreMesh(core_axis_name="c", subcore_axis_name="s")
    out_type = [jax.ShapeDtypeStruct((NC, NP, F), jnp.float32)]
    scratch = [
        pltpu.VMEM((CH, K), jnp.int32),
        pltpu.VMEM((CH, K), jnp.int32),
        pltpu.VMEM((K, F), jnp.float32),
        pltpu.VMEM((ZR, F), jnp.float32),
        pltpu.VMEM_SHARED((NP, F), jnp.float32),
        pltpu.SemaphoreType.DMA,
    ]
    if with_deg:
        out_type.append(jax.ShapeDtypeStruct((NC, NP), jnp.float32))
        scratch += [
            pltpu.VMEM((128,), jnp.float32),
            pltpu.VMEM((RT,), jnp.float32),
            pltpu.VMEM_SHARED((NP,), jnp.float32),
        ]

    def body(table_hbm, src_hbm, dst_hbm, *rest):
        if with_deg:
            (agg_out, deg_out, src_v, dst_v, rows_v, zb, agg_sh, sem,
             ones_v, zdeg, deg_sh) = rest
        else:
            agg_out, src_v, dst_v, rows_v, zb, agg_sh, sem = rest
        cid = lax.axis_index("c")
        sid = lax.axis_index("s")
        wid = sid * NC + cid

        zeros16 = jnp.zeros((16,), jnp.float32)

        def zrow(r, c):
            for cc in range(F // 16):
                zb[r, pl.ds(cc * 16, 16)] = zeros16
            return c
        lax.fori_loop(0, ZR, zrow, 0)
        row0 = sid * RT
        for b in range(RT // ZR):
            pltpu.sync_copy(zb, agg_sh.at[pl.ds(row0 + b * ZR, ZR)])
        if with_deg:
            def zdrow(r, c):
                zdeg[pl.ds(r * 16, 16)] = zeros16
                return c
            lax.fori_loop(0, RT // 16, zdrow, 0)
            pltpu.sync_copy(zdeg, deg_sh.at[pl.ds(row0, RT)])
            ones16 = jnp.ones((16,), jnp.float32)
            for i in range(128 // 16):
                ones_v[pl.ds(i * 16, 16)] = ones16
        plsc.subcore_barrier()

        pltpu.sync_copy(src_hbm.at[pl.ds(wid * CH, CH)], src_v)
        pltpu.sync_copy(dst_hbm.at[pl.ds(wid * CH, CH)], dst_v)

        def chunk(j, c):
            pltpu.async_copy(table_hbm.at[src_v.at[j]], rows_v, sem).wait()
            pltpu.sync_copy(rows_v, agg_sh.at[dst_v.at[j]], add=True)
            if with_deg:
                pltpu.sync_copy(ones_v.at[pl.ds(0, K)],
                                deg_sh.at[dst_v.at[j]], add=True)
            return c
        lax.fori_loop(0, CH, chunk, 0)

        plsc.subcore_barrier()

        pltpu.sync_copy(agg_sh.at[pl.ds(row0, RT)],
                        agg_out.at[cid, pl.ds(row0, RT)])
        if with_deg:
            pltpu.sync_copy(deg_sh.at[pl.ds(row0, RT)],
                            deg_out.at[cid, pl.ds(row0, RT)])

    fn = functools.partial(pl.kernel, mesh=mesh, out_type=out_type,
                           scratch_types=scratch)(body)
    return fn(table, src2d, dst2d)


def _tc_layer1_body(x_ref, a_ref, d_ref, ws_ref, wn_ref, b_ref, wn2_ref,
                    h_ref, p_ref):
    x = x_ref[...]
    a = a_ref[0] + a_ref[1]
    dinv = 1.0 / jnp.maximum(d_ref[0] + d_ref[1], 1.0)
    hn = jnp.dot(a, wn_ref[...], preferred_element_type=jnp.float32)
    h = (jnp.dot(x, ws_ref[...], preferred_element_type=jnp.float32)
         + dinv * hn + b_ref[...])
    h = jnp.maximum(h, 0.0)
    h_ref[...] = h
    p_ref[...] = jnp.dot(h, wn2_ref[...], preferred_element_type=jnp.float32)


def _tc_layer1(x, aggp, deg3, W_self1, W_neigh1, b1r, W_neigh2):
    return pl.pallas_call(
        _tc_layer1_body,
        grid=(GRID,),
        in_specs=[
            pl.BlockSpec((RB, D), lambda i: (i, 0)),
            pl.BlockSpec((NC, RB, D), lambda i: (0, i, 0)),
            pl.BlockSpec((NC, RB, 1), lambda i: (0, i, 0)),
            pl.BlockSpec((D, H), lambda i: (0, 0)),
            pl.BlockSpec((D, H), lambda i: (0, 0)),
            pl.BlockSpec((1, H), lambda i: (0, 0)),
            pl.BlockSpec((H, PF), lambda i: (0, 0)),
        ],
        out_specs=[
            pl.BlockSpec((RB, H), lambda i: (i, 0)),
            pl.BlockSpec((RB, PF), lambda i: (i, 0)),
        ],
        out_shape=[
            jax.ShapeDtypeStruct((N, H), jnp.float32),
            jax.ShapeDtypeStruct((N, PF), jnp.float32),
        ],
    )(x, aggp, deg3, W_self1, W_neigh1, b1r, W_neigh2)


def _tc_layer2_body(h_ref, a_ref, d_ref, ws_ref, b_ref, o_ref):
    a = a_ref[0, :, :C] + a_ref[1, :, :C]
    dinv = 1.0 / jnp.maximum(d_ref[0] + d_ref[1], 1.0)
    o_ref[...] = (jnp.dot(h_ref[...], ws_ref[...],
                          preferred_element_type=jnp.float32)
                  + dinv * a + b_ref[...])


def _tc_layer2(h, agg2p, deg3, W_self2, b2r):
    return pl.pallas_call(
        _tc_layer2_body,
        grid=(GRID,),
        in_specs=[
            pl.BlockSpec((RB, H), lambda i: (i, 0)),
            pl.BlockSpec((NC, RB, PF), lambda i: (0, i, 0)),
            pl.BlockSpec((NC, RB, 1), lambda i: (0, i, 0)),
            pl.BlockSpec((H, C), lambda i: (0, 0)),
            pl.BlockSpec((1, C), lambda i: (0, 0)),
        ],
        out_specs=pl.BlockSpec((RB, C), lambda i: (i, 0)),
        out_shape=jax.ShapeDtypeStruct((N, C), jnp.float32),
    )(h, agg2p, deg3, W_self2, b2r)


def kernel(features, edge_index, W_self1, W_neigh1, b1, W_self2, W_neigh2,
           b2):
    src2d = edge_index[0].reshape(ER, K)
    dst2d = edge_index[1].reshape(ER, K)

    agg1p, degp = _sc_agg(features, src2d, dst2d, D, with_deg=True)
    deg3 = degp.reshape(NC, NP, 1)

    wn2p = jnp.pad(W_neigh2, ((0, 0), (0, PF - C)))
    h, p = _tc_layer1(features, agg1p, deg3, W_self1, W_neigh1,
                      b1.reshape(1, H), wn2p)

    (agg2p,) = _sc_agg(p, src2d, dst2d, PF, with_deg=False)

    return _tc_layer2(h, agg2p, deg3, W_self2, b2.reshape(1, C))


# speedup vs baseline: 14.8695x; 1.4331x over previous
import functools

import jax
import jax.numpy as jnp
from jax import lax
from jax.experimental import pallas as pl
from jax.experimental.pallas import tpu as pltpu
from jax.experimental.pallas import tpu_sc as plsc

N = 10000
E = 320000
D = 128
H = 256
C = 32

NP = 10240
K = 125
ER = E // K
NC = 2
NS = 16
NW = NC * NS
CH = ER // NW
RT = NP // NS
ZR = 32
SB = 40

RB = 1000
GRID = N // RB
PF = 128


def _sc_agg(table, src2d, dst2d, F, with_deg):
    mesh = plsc.VectorSubcoreMesh(core_axis_name="c", subcore_axis_name="s")
    out_type = [jax.ShapeDtypeStruct((NC, NP, F), jnp.float32)]
    scratch = [
        pltpu.VMEM((SB, K), jnp.int32),
        pltpu.VMEM((SB, K), jnp.int32),
        pltpu.VMEM((K, F), jnp.float32),
        pltpu.VMEM((K, F), jnp.float32),
        pltpu.VMEM((ZR, F), jnp.float32),
        pltpu.VMEM_SHARED((NP, F), jnp.float32),
        pltpu.SemaphoreType.DMA,
        pltpu.SemaphoreType.DMA,
    ]
    if with_deg:
        out_type.append(jax.ShapeDtypeStruct((NC, NP), jnp.float32))
        scratch += [
            pltpu.VMEM((128,), jnp.float32),
            pltpu.VMEM((RT,), jnp.float32),
            pltpu.VMEM_SHARED((NP,), jnp.float32),
        ]

    def body(table_hbm, src_hbm, dst_hbm, *rest):
        if with_deg:
            (agg_out, deg_out, src_v, dst_v, rows0, rows1, zb, agg_sh,
             sem0, sem1, ones_v, zdeg, deg_sh) = rest
        else:
            (agg_out, src_v, dst_v, rows0, rows1, zb, agg_sh,
             sem0, sem1) = rest
        rows_bufs = (rows0, rows1)
        sems = (sem0, sem1)
        cid = lax.axis_index("c")
        sid = lax.axis_index("s")
        wid = sid * NC + cid

        zeros16 = jnp.zeros((16,), jnp.float32)

        def zrow(r, c):
            for cc in range(F // 16):
                zb[r, pl.ds(cc * 16, 16)] = zeros16
            return c
        lax.fori_loop(0, ZR, zrow, 0)
        row0 = sid * RT
        for b in range(RT // ZR):
            pltpu.sync_copy(zb, agg_sh.at[pl.ds(row0 + b * ZR, ZR)])
        if with_deg:
            def zdrow(r, c):
                zdeg[pl.ds(r * 16, 16)] = zeros16
                return c
            lax.fori_loop(0, RT // 16, zdrow, 0)
            pltpu.sync_copy(zdeg, deg_sh.at[pl.ds(row0, RT)])
            ones16 = jnp.ones((16,), jnp.float32)
            for i in range(128 // 16):
                ones_v[pl.ds(i * 16, 16)] = ones16
        plsc.subcore_barrier()

        for s in range(CH // SB):
            pltpu.sync_copy(src_hbm.at[pl.ds(wid * CH + s * SB, SB)], src_v)
            pltpu.sync_copy(dst_hbm.at[pl.ds(wid * CH + s * SB, SB)], dst_v)
            pltpu.async_copy(table_hbm.at[src_v.at[0]], rows0, sem0)
            pltpu.async_copy(table_hbm.at[src_v.at[1]], rows1, sem1)

            def chunk2(g, c):
                for b in range(2):
                    j = 2 * g + b
                    rows, sem = rows_bufs[b], sems[b]
                    pltpu.make_async_copy(table_hbm.at[src_v.at[j]], rows,
                                          sem).wait()
                    pltpu.sync_copy(rows, agg_sh.at[dst_v.at[j]], add=True)
                    if with_deg:
                        pltpu.sync_copy(ones_v.at[pl.ds(0, K)],
                                        deg_sh.at[dst_v.at[j]], add=True)
                    jn = jnp.where(j + 2 < SB, j + 2, j)
                    pltpu.async_copy(table_hbm.at[src_v.at[jn]], rows, sem)
                return c
            lax.fori_loop(0, SB // 2, chunk2, 0)
            for b in range(2):
                pltpu.make_async_copy(table_hbm.at[src_v.at[0]],
                                      rows_bufs[b], sems[b]).wait()

        plsc.subcore_barrier()

        pltpu.sync_copy(agg_sh.at[pl.ds(row0, RT)],
                        agg_out.at[cid, pl.ds(row0, RT)])
        if with_deg:
            pltpu.sync_copy(deg_sh.at[pl.ds(row0, RT)],
                            deg_out.at[cid, pl.ds(row0, RT)])

    fn = functools.partial(pl.kernel, mesh=mesh, out_type=out_type,
                           scratch_types=scratch)(body)
    return fn(table, src2d, dst2d)


def _tc_layer1_body(x_ref, a_ref, d_ref, ws_ref, wn_ref, b_ref, wn2_ref,
                    h_ref, p_ref):
    x = x_ref[...]
    a = a_ref[0] + a_ref[1]
    dinv = 1.0 / jnp.maximum(d_ref[0] + d_ref[1], 1.0)
    hn = jnp.dot(a, wn_ref[...], preferred_element_type=jnp.float32)
    h = (jnp.dot(x, ws_ref[...], preferred_element_type=jnp.float32)
         + dinv * hn + b_ref[...])
    h = jnp.maximum(h, 0.0)
    h_ref[...] = h
    p_ref[...] = jnp.dot(h, wn2_ref[...], preferred_element_type=jnp.float32)


def _tc_layer1(x, aggp, deg3, W_self1, W_neigh1, b1r, W_neigh2):
    return pl.pallas_call(
        _tc_layer1_body,
        grid=(GRID,),
        in_specs=[
            pl.BlockSpec((RB, D), lambda i: (i, 0)),
            pl.BlockSpec((NC, RB, D), lambda i: (0, i, 0)),
            pl.BlockSpec((NC, RB, 1), lambda i: (0, i, 0)),
            pl.BlockSpec((D, H), lambda i: (0, 0)),
            pl.BlockSpec((D, H), lambda i: (0, 0)),
            pl.BlockSpec((1, H), lambda i: (0, 0)),
            pl.BlockSpec((H, PF), lambda i: (0, 0)),
        ],
        out_specs=[
            pl.BlockSpec((RB, H), lambda i: (i, 0)),
            pl.BlockSpec((RB, PF), lambda i: (i, 0)),
        ],
        out_shape=[
            jax.ShapeDtypeStruct((N, H), jnp.float32),
            jax.ShapeDtypeStruct((N, PF), jnp.float32),
        ],
    )(x, aggp, deg3, W_self1, W_neigh1, b1r, W_neigh2)


def _tc_layer2_body(h_ref, a_ref, d_ref, ws_ref, b_ref, o_ref):
    a = a_ref[0, :, :C] + a_ref[1, :, :C]
    dinv = 1.0 / jnp.maximum(d_ref[0] + d_ref[1], 1.0)
    o_ref[...] = (jnp.dot(h_ref[...], ws_ref[...],
                          preferred_element_type=jnp.float32)
                  + dinv * a + b_ref[...])


def _tc_layer2(h, agg2p, deg3, W_self2, b2r):
    return pl.pallas_call(
        _tc_layer2_body,
        grid=(GRID,),
        in_specs=[
            pl.BlockSpec((RB, H), lambda i: (i, 0)),
            pl.BlockSpec((NC, RB, PF), lambda i: (0, i, 0)),
            pl.BlockSpec((NC, RB, 1), lambda i: (0, i, 0)),
            pl.BlockSpec((H, C), lambda i: (0, 0)),
            pl.BlockSpec((1, C), lambda i: (0, 0)),
        ],
        out_specs=pl.BlockSpec((RB, C), lambda i: (i, 0)),
        out_shape=jax.ShapeDtypeStruct((N, C), jnp.float32),
    )(h, agg2p, deg3, W_self2, b2r)


def kernel(features, edge_index, W_self1, W_neigh1, b1, W_self2, W_neigh2,
           b2):
    src2d = edge_index[0].reshape(ER, K)
    dst2d = edge_index[1].reshape(ER, K)

    agg1p, degp = _sc_agg(features, src2d, dst2d, D, with_deg=True)
    deg3 = degp.reshape(NC, NP, 1)

    wn2p = jnp.pad(W_neigh2, ((0, 0), (0, PF - C)))
    h, p = _tc_layer1(features, agg1p, deg3, W_self1, W_neigh1,
                      b1.reshape(1, H), wn2p)

    (agg2p,) = _sc_agg(p, src2d, dst2d, PF, with_deg=False)

    return _tc_layer2(h, agg2p, deg3, W_self2, b2.reshape(1, C))
